# Initial kernel scaffold; baseline (speedup 1.0000x reference)
import jax
import jax.numpy as jnp
from jax.experimental import pallas as pl


def kernel(user_table, item_table, feat0, feat1, feat2, enc0_W, enc0_b, enc0_g, enc0_beta, enc1_W, enc1_b, enc1_g, enc1_beta, enc2_W, enc2_b, enc2_g, enc2_beta, att_W1, att_b1, att_W2, att_b2, adj_row, adj_col, adj_val):
    raise NotImplementedError("write your pallas kernel here")


# speedup vs baseline: 1.7383x; 1.7383x over previous
import functools

import jax
import jax.numpy as jnp
from jax import lax
from jax.experimental import pallas as pl
from jax.experimental.pallas import tpu as pltpu
from jax.experimental.pallas import tpu_sc as plsc

NU = 25000
NI = 25000
NN = NU + NI
D = 64
E = 800000

NC = 2
NS = 16
HALF = 25000
PAD = 88
HALF_PAD = HALF + PAD
NPAD = 2 * HALF_PAD
RPT = HALF_PAD // NS

EPC = 128
GPC = 2
CHUNK = EPC * GPC
E_PAD = 819200
EPT = E_PAD // NS
NCHUNKS = EPT // CHUNK

ZR = 56

_f32 = jnp.float32
_i32 = jnp.int32


UB = 1000
IB = 1000


def _ln_leaky(h, g, beta):
    mu = jnp.mean(h, axis=-1, keepdims=True)
    var = jnp.mean((h - mu) ** 2, axis=-1, keepdims=True)
    h = (h - mu) * lax.rsqrt(var + 1e-5) * g + beta
    return jnp.where(h >= 0, h, 0.2 * h)


def _user_stats_body(feat2, utab, W, b, g, beta, m2sum, usum):
    i = pl.program_id(0)
    h = jnp.dot(feat2[...], W[...], preferred_element_type=_f32) + b[...]
    h = _ln_leaky(h, g[...], beta[...])

    @pl.when(i == 0)
    def _():
        m2sum[...] = jnp.zeros_like(m2sum)
        usum[...] = jnp.zeros_like(usum)

    m2sum[...] += jnp.sum(h, axis=0, keepdims=True)
    usum[...] += jnp.sum(utab[...], axis=0, keepdims=True)


def _user_stats(feat2, utab, W, b, g, beta):
    grid = (NU // UB,)
    full = lambda *s: pl.BlockSpec(s, lambda i: (0,) * len(s))
    return pl.pallas_call(
        _user_stats_body,
        grid=grid,
        in_specs=[
            pl.BlockSpec((UB, 128), lambda i: (i, 0)),
            pl.BlockSpec((UB, D), lambda i: (i, 0)),
            full(128, D), full(1, D), full(1, D), full(1, D),
        ],
        out_specs=(full(1, D), full(1, D)),
        out_shape=(jax.ShapeDtypeStruct((1, D), _f32),
                   jax.ShapeDtypeStruct((1, D), _f32)),
    )(feat2, utab, W, b, g, beta)


def _item_fuse_body(feat0, feat1, itab,
                    W0, b0, g0, be0, W1, b1, g1, be1,
                    aW1u, aW1i, ab1, aW2, ab2, m2sum, usum, out):
    m0 = _ln_leaky(jnp.dot(feat0[...], W0[...], preferred_element_type=_f32)
                   + b0[...], g0[...], be0[...])
    m1 = _ln_leaky(jnp.dot(feat1[...], W1[...], preferred_element_type=_f32)
                   + b1[...], g1[...], be1[...])
    m2m = m2sum[...] * (1.0 / NU)
    um = usum[...] * (1.0 / NU)
    it = itab[...]
    h = jnp.tanh(jnp.dot(it, aW1i[...], preferred_element_type=_f32)
                 + jnp.dot(um, aW1u[...], preferred_element_type=_f32)
                 + ab1[...])
    logits = jnp.dot(h, aW2[...], preferred_element_type=_f32) + ab2[...]
    mx = jnp.max(logits, axis=-1, keepdims=True)
    ex = jnp.exp(logits - mx)
    sm = ex / jnp.sum(ex, axis=-1, keepdims=True)
    out[...] = (it + sm[:, 0:1] * m0 + sm[:, 1:2] * m1 + sm[:, 2:3] * m2m)


def _item_fuse(feat0, feat1, itab, W0, b0, g0, be0, W1, b1, g1, be1,
               aW1u, aW1i, ab1, aW2, ab2, m2sum, usum):
    grid = (NI // IB,)
    full = lambda *s: pl.BlockSpec(s, lambda i: (0,) * len(s))
    return pl.pallas_call(
        _item_fuse_body,
        grid=grid,
        in_specs=[
            pl.BlockSpec((IB, 512), lambda i: (i, 0)),
            pl.BlockSpec((IB, 384), lambda i: (i, 0)),
            pl.BlockSpec((IB, D), lambda i: (i, 0)),
            full(512, D), full(1, D), full(1, D), full(1, D),
            full(384, D), full(1, D), full(1, D), full(1, D),
            full(D, D), full(D, D), full(1, D),
            full(D, 128), full(1, 128),
            full(1, D), full(1, D),
        ],
        out_specs=pl.BlockSpec((IB, D), lambda i: (i, 0)),
        out_shape=jax.ShapeDtypeStruct((NI, D), _f32),
    )(feat0, feat1, itab, W0, b0, g0, be0, W1, b1, g1, be1,
      aW1u, aW1i, ab1, aW2, ab2, m2sum, usum)


def _mean3_body(a, b, c, out):
    out[...] = (a[...] + b[...] + c[...]) * (1.0 / 3.0)


def _mean3(a, b, c):
    grid = (a.shape[0] // UB,)
    return pl.pallas_call(
        _mean3_body,
        grid=grid,
        in_specs=[pl.BlockSpec((UB, D), lambda i: (i, 0))] * 3,
        out_specs=pl.BlockSpec((UB, D), lambda i: (i, 0)),
        out_shape=jax.ShapeDtypeStruct(a.shape, _f32),
    )(a, b, c)


def _prop_body(emb, rows2d, cols2d, vals1d, out,
               acc, ridx, cidx, valv, rbuf, zbuf, sem):
    c = lax.axis_index("c")
    s = lax.axis_index("s")
    base = c * HALF

    zv = jnp.zeros((16,), _f32)

    def zb(i, carry):
        for q in range(D // 16):
            zbuf[i, pl.ds(16 * q, 16)] = zv
        return carry

    lax.fori_loop(0, ZR, zb, 0)

    def za(z, carry):
        pltpu.sync_copy(zbuf, acc.at[pl.ds(s * RPT + z * ZR, ZR)])
        return carry

    lax.fori_loop(0, RPT // ZR, za, 0)
    plsc.subcore_barrier()

    def chunk(k, carry):
        g0 = pl.multiple_of((s * EPT + k * CHUNK) // EPC, GPC)
        pltpu.sync_copy(rows2d.at[pl.ds(g0, GPC)], ridx)
        pltpu.sync_copy(cols2d.at[pl.ds(g0, GPC)], cidx)
        eoff = pl.multiple_of(s * EPT + k * CHUNK, CHUNK)
        pltpu.sync_copy(vals1d.at[pl.ds(eoff, CHUNK)], valv)
        for g in range(GPC):
            for q in range(EPC // 16):
                sl = pl.ds(16 * q, 16)
                cid = cidx[g, sl]
                cidx[g, sl] = jnp.where(cid >= HALF, cid + PAD, cid)
                rid = ridx[g, sl]
                inh = (rid >= base) & (rid < base + HALF)
                ridx[g, sl] = jnp.where(inh, rid - base, HALF + s)
                fl = pl.ds(g * EPC + 16 * q, 16)
                vv = valv[fl]
                valv[fl] = jnp.where(inh, vv, jnp.zeros((16,), _f32))
        cps = [pltpu.async_copy(emb.at[cidx.at[g]],
                                rbuf.at[pl.ds(g * EPC, EPC)], sem)
               for g in range(GPC)]
        for cp in cps:
            cp.wait()

        def mul(j16, carry):
            jb = j16 * 16
            v16 = valv[pl.ds(jb, 16)]
            for i in range(16):
                vi = v16[i]
                for q in range(D // 16):
                    sl = pl.ds(16 * q, 16)
                    rbuf[jb + i, sl] = rbuf[jb + i, sl] * vi
            return carry

        lax.fori_loop(0, CHUNK // 16, mul, 0)

        for g in range(GPC):
            pltpu.sync_copy(rbuf.at[pl.ds(g * EPC, EPC)],
                            acc.at[ridx.at[g]], add=True)
        return carry

    lax.fori_loop(0, NCHUNKS, chunk, 0)
    plsc.subcore_barrier()

    pltpu.sync_copy(acc.at[pl.ds(s * RPT, RPT)],
                    out.at[pl.ds(c * HALF_PAD + s * RPT, RPT)])


_propagate = functools.partial(
    pl.kernel,
    out_type=jax.ShapeDtypeStruct((NPAD, D), _f32),
    mesh=plsc.VectorSubcoreMesh(core_axis_name="c", subcore_axis_name="s"),
    compiler_params=pltpu.CompilerParams(use_tc_tiling_on_sc=False),
    scratch_types=[
        pltpu.VMEM_SHARED((HALF_PAD, D), _f32),
        pltpu.VMEM((GPC, EPC), _i32),
        pltpu.VMEM((GPC, EPC), _i32),
        pltpu.VMEM((CHUNK,), _f32),
        pltpu.VMEM((CHUNK, D), _f32),
        pltpu.VMEM((ZR, D), _f32),
        pltpu.SemaphoreType.DMA,
    ],
)(_prop_body)


def kernel(user_table, item_table, feat0, feat1, feat2,
           enc0_W, enc0_b, enc0_g, enc0_beta,
           enc1_W, enc1_b, enc1_g, enc1_beta,
           enc2_W, enc2_b, enc2_g, enc2_beta,
           att_W1, att_b1, att_W2, att_b2,
           adj_row, adj_col, adj_val):
    r1 = lambda v: v.reshape(1, -1)
    m2sum, usum = _user_stats(feat2, user_table, enc2_W, r1(enc2_b),
                              r1(enc2_g), r1(enc2_beta))
    aW2 = jnp.pad(att_W2, ((0, 0), (0, 128 - 3)))
    ab2 = jnp.pad(att_b2, (0, 128 - 3), constant_values=-1e30).reshape(1, -1)
    item_emb = _item_fuse(
        feat0, feat1, item_table,
        enc0_W, r1(enc0_b), r1(enc0_g), r1(enc0_beta),
        enc1_W, r1(enc1_b), r1(enc1_g), r1(enc1_beta),
        att_W1[:D], att_W1[D:], r1(att_b1), aW2, ab2, m2sum, usum)

    z = jnp.zeros((PAD, D), _f32)
    e0 = jnp.concatenate([user_table, z, item_emb, z], axis=0)

    npad = E_PAD - E
    rows2d = jnp.pad(adj_row.astype(_i32), (0, npad)).reshape(-1, EPC)
    cols2d = jnp.pad(adj_col.astype(_i32), (0, npad)).reshape(-1, EPC)
    vals1d = jnp.pad(adj_val, (0, npad))

    e1 = _propagate(e0, rows2d, cols2d, vals1d)
    e2 = _propagate(e1, rows2d, cols2d, vals1d)

    u_out = _mean3(user_table, e1[:NU], e2[:NU])
    i_out = _mean3(item_emb, e1[HALF_PAD:HALF_PAD + NI],
                   e2[HALF_PAD:HALF_PAD + NI])
    return u_out, i_out


# speedup vs baseline: 2.2704x; 1.3061x over previous
import functools

import jax
import jax.numpy as jnp
from jax import lax
from jax.experimental import pallas as pl
from jax.experimental.pallas import tpu as pltpu
from jax.experimental.pallas import tpu_sc as plsc

NU = 25000
NI = 25000
NN = NU + NI
D = 64
E = 800000

NC = 2
NS = 16
HALF = 25000
PAD = 88
HALF_PAD = HALF + PAD
NPAD = 2 * HALF_PAD
RPT = HALF_PAD // NS

EPC = 128
GPS = 8
SUPER = EPC * GPS
E_PAD = 819200
EPT = E_PAD // NS
NSUP = EPT // SUPER
E_IDX = E_PAD + 2 * SUPER

_f32 = jnp.float32
_i32 = jnp.int32


UB = 1000
IB = 1000


def _ln_leaky(h, g, beta):
    mu = jnp.mean(h, axis=-1, keepdims=True)
    var = jnp.mean((h - mu) ** 2, axis=-1, keepdims=True)
    h = (h - mu) * lax.rsqrt(var + 1e-5) * g + beta
    return jnp.where(h >= 0, h, 0.2 * h)


def _user_stats_body(feat2, utab, W, b, g, beta, m2sum, usum):
    i = pl.program_id(0)
    h = jnp.dot(feat2[...], W[...], preferred_element_type=_f32) + b[...]
    h = _ln_leaky(h, g[...], beta[...])

    @pl.when(i == 0)
    def _():
        m2sum[...] = jnp.zeros_like(m2sum)
        usum[...] = jnp.zeros_like(usum)

    m2sum[...] += jnp.sum(h, axis=0, keepdims=True)
    usum[...] += jnp.sum(utab[...], axis=0, keepdims=True)


def _user_stats(feat2, utab, W, b, g, beta):
    grid = (NU // UB,)
    full = lambda *s: pl.BlockSpec(s, lambda i: (0,) * len(s))
    return pl.pallas_call(
        _user_stats_body,
        grid=grid,
        in_specs=[
            pl.BlockSpec((UB, 128), lambda i: (i, 0)),
            pl.BlockSpec((UB, D), lambda i: (i, 0)),
            full(128, D), full(1, D), full(1, D), full(1, D),
        ],
        out_specs=(full(1, D), full(1, D)),
        out_shape=(jax.ShapeDtypeStruct((1, D), _f32),
                   jax.ShapeDtypeStruct((1, D), _f32)),
    )(feat2, utab, W, b, g, beta)


def _item_fuse_body(feat0, feat1, itab,
                    W0, b0, g0, be0, W1, b1, g1, be1,
                    aW1u, aW1i, ab1, aW2, ab2, m2sum, usum, out):
    m0 = _ln_leaky(jnp.dot(feat0[...], W0[...], preferred_element_type=_f32)
                   + b0[...], g0[...], be0[...])
    m1 = _ln_leaky(jnp.dot(feat1[...], W1[...], preferred_element_type=_f32)
                   + b1[...], g1[...], be1[...])
    m2m = m2sum[...] * (1.0 / NU)
    um = usum[...] * (1.0 / NU)
    it = itab[...]
    h = jnp.tanh(jnp.dot(it, aW1i[...], preferred_element_type=_f32)
                 + jnp.dot(um, aW1u[...], preferred_element_type=_f32)
                 + ab1[...])
    logits = jnp.dot(h, aW2[...], preferred_element_type=_f32) + ab2[...]
    mx = jnp.max(logits, axis=-1, keepdims=True)
    ex = jnp.exp(logits - mx)
    sm = ex / jnp.sum(ex, axis=-1, keepdims=True)
    out[...] = (it + sm[:, 0:1] * m0 + sm[:, 1:2] * m1 + sm[:, 2:3] * m2m)


def _item_fuse(feat0, feat1, itab, W0, b0, g0, be0, W1, b1, g1, be1,
               aW1u, aW1i, ab1, aW2, ab2, m2sum, usum):
    grid = (NI // IB,)
    full = lambda *s: pl.BlockSpec(s, lambda i: (0,) * len(s))
    return pl.pallas_call(
        _item_fuse_body,
        grid=grid,
        in_specs=[
            pl.BlockSpec((IB, 512), lambda i: (i, 0)),
            pl.BlockSpec((IB, 384), lambda i: (i, 0)),
            pl.BlockSpec((IB, D), lambda i: (i, 0)),
            full(512, D), full(1, D), full(1, D), full(1, D),
            full(384, D), full(1, D), full(1, D), full(1, D),
            full(D, D), full(D, D), full(1, D),
            full(D, 128), full(1, 128),
            full(1, D), full(1, D),
        ],
        out_specs=pl.BlockSpec((IB, D), lambda i: (i, 0)),
        out_shape=jax.ShapeDtypeStruct((NI, D), _f32),
    )(feat0, feat1, itab, W0, b0, g0, be0, W1, b1, g1, be1,
      aW1u, aW1i, ab1, aW2, ab2, m2sum, usum)


def _mean3_body(a, b, c, out):
    out[...] = (a[...] + b[...] + c[...]) * (1.0 / 3.0)


def _mean3(a, b, c):
    grid = (a.shape[0] // UB,)
    return pl.pallas_call(
        _mean3_body,
        grid=grid,
        in_specs=[pl.BlockSpec((UB, D), lambda i: (i, 0))] * 3,
        out_specs=pl.BlockSpec((UB, D), lambda i: (i, 0)),
        out_shape=jax.ShapeDtypeStruct(a.shape, _f32),
    )(a, b, c)


def _prop_body(emb, rows2d, cols2d, vals1d, out,
               acc, ridx0, cidx0, valv0, ridx1, cidx1, valv1,
               rbuf0, rbuf1, sidx,
               semI0, semI1, semG0, semG1, semS0, semS1):
    c = lax.axis_index("c")
    s = lax.axis_index("s")
    base = c * HALF
    rbufs = (rbuf0, rbuf1)
    semG = (semG0, semG1)
    semS = (semS0, semS1)
    slots = ((ridx0, cidx0, valv0, semI0), (ridx1, cidx1, valv1, semI1))
    zv = jnp.zeros((16,), _f32)
    civ = lax.iota(_i32, 16)
    ebase = s * EPT

    def zb(i, carry):
        for q in range(D // 16):
            rbuf0[i, pl.ds(16 * q, 16)] = zv
            rbuf1[i, pl.ds(16 * q, 16)] = zv
        return carry

    lax.fori_loop(0, EPC, zb, 0)
    for b in range(2):
        def sin(q, carry, _b=b):
            sidx[_b, pl.ds(16 * q, 16)] = jnp.full((16,), HALF, _i32) + civ
            return carry
        lax.fori_loop(0, EPC // 16, sin, 0)

    nfull = RPT // EPC
    rem = RPT - nfull * EPC
    for z in range(nfull):
        pltpu.sync_copy(rbuf0, acc.at[pl.ds(s * RPT + z * EPC, EPC)])
    if rem:
        pltpu.sync_copy(rbuf0.at[pl.ds(0, rem)],
                        acc.at[pl.ds(s * RPT + nfull * EPC, rem)])
    plsc.subcore_barrier()

    def issue_idx(m, slot):
        ridx_r, cidx_r, valv_r, semI = slot
        g0 = pl.multiple_of((ebase + m * SUPER) // EPC, GPS)
        e0_ = pl.multiple_of(ebase + m * SUPER, SUPER)
        pltpu.async_copy(rows2d.at[pl.ds(g0, GPS)], ridx_r, semI)
        pltpu.async_copy(cols2d.at[pl.ds(g0, GPS)], cidx_r, semI)
        pltpu.async_copy(vals1d.at[pl.ds(e0_, SUPER)], valv_r, semI)

    def wait_idx(m, slot):
        ridx_r, cidx_r, valv_r, semI = slot
        g0 = pl.multiple_of((ebase + m * SUPER) // EPC, GPS)
        e0_ = pl.multiple_of(ebase + m * SUPER, SUPER)
        pltpu.make_async_copy(rows2d.at[pl.ds(g0, GPS)], ridx_r, semI).wait()
        pltpu.make_async_copy(cols2d.at[pl.ds(g0, GPS)], cidx_r, semI).wait()
        pltpu.make_async_copy(vals1d.at[pl.ds(e0_, SUPER)], valv_r, semI).wait()

    def remap_copy(slot, g, b):
        ridx_r, cidx_r, valv_r, _ = slot

        def rq(q, carry):
            sl = pl.ds(16 * q, 16)
            cid = cidx_r[g, sl]
            cidx_r[g, sl] = jnp.where(cid >= HALF, cid + PAD, cid)
            rid = ridx_r[g, sl]
            inh = (rid >= base) & (rid < base + HALF)
            dummy = jnp.full((16,), HALF, _i32) + 16 * (q & 3) + civ
            lloc = jnp.where(inh, rid - base, dummy)
            ridx_r[g, sl] = lloc
            sidx[b, sl] = lloc
            fl = pl.ds(g * EPC + 16 * q, 16)
            vv = valv_r[fl]
            valv_r[fl] = jnp.where(inh, vv, zv)
            return carry

        lax.fori_loop(0, EPC // 16, rq, 0)

    def mul(slot, g, b):
        valv_r = slot[2]
        rb = rbufs[b]

        def mbody(j16, carry):
            jb = j16 * 16
            v16 = valv_r[pl.ds(g * EPC + jb, 16)]
            for i in range(16):
                vi = v16[i]
                for q in range(D // 16):
                    sl2 = pl.ds(16 * q, 16)
                    rb[jb + i, sl2] = rb[jb + i, sl2] * vi
            return carry

        lax.fori_loop(0, EPC // 16, mbody, 0)

    def do_super(m, slot, cs):
        wait_idx(m, slot)
        cg = [None, None]
        for g in (0, 1):
            b = g & 1
            if cs[b] is not None:
                cs[b].wait()
            remap_copy(slot, g, b)
            cg[b] = pltpu.async_copy(emb.at[slot[1].at[g]], rbufs[b], semG[b])
        for g in range(GPS):
            b = g & 1
            cg[b].wait()
            mul(slot, g, b)
            cs[b] = pltpu.async_copy(rbufs[b], acc.at[sidx.at[b]], semS[b],
                                     add=True)
            if g + 2 < GPS:
                cs[b].wait()
                cs[b] = None
                remap_copy(slot, g + 2, b)
                cg[b] = pltpu.async_copy(emb.at[slot[1].at[g + 2]],
                                         rbufs[b], semG[b])
        issue_idx(m + 2, slot)
        return cs

    issue_idx(0, slots[0])
    issue_idx(1, slots[1])

    def body(i, carry):
        cs = do_super(2 * i, slots[0], [None, None])
        cs = do_super(2 * i + 1, slots[1], cs)
        cs[0].wait()
        cs[1].wait()
        return carry

    lax.fori_loop(0, NSUP // 2, body, 0)

    wait_idx(NSUP, slots[0])
    wait_idx(NSUP + 1, slots[1])
    plsc.subcore_barrier()
    pltpu.sync_copy(acc.at[pl.ds(s * RPT, RPT)],
                    out.at[pl.ds(c * HALF_PAD + s * RPT, RPT)])


_propagate = functools.partial(
    pl.kernel,
    out_type=jax.ShapeDtypeStruct((NPAD, D), _f32),
    mesh=plsc.VectorSubcoreMesh(core_axis_name="c", subcore_axis_name="s"),
    compiler_params=pltpu.CompilerParams(use_tc_tiling_on_sc=False),
    scratch_types=[
        pltpu.VMEM_SHARED((HALF_PAD, D), _f32),
        pltpu.VMEM((GPS, EPC), _i32),
        pltpu.VMEM((GPS, EPC), _i32),
        pltpu.VMEM((SUPER,), _f32),
        pltpu.VMEM((GPS, EPC), _i32),
        pltpu.VMEM((GPS, EPC), _i32),
        pltpu.VMEM((SUPER,), _f32),
        pltpu.VMEM((EPC, D), _f32),
        pltpu.VMEM((EPC, D), _f32),
        pltpu.VMEM((2, EPC), _i32),
        pltpu.SemaphoreType.DMA,
        pltpu.SemaphoreType.DMA,
        pltpu.SemaphoreType.DMA,
        pltpu.SemaphoreType.DMA,
        pltpu.SemaphoreType.DMA,
        pltpu.SemaphoreType.DMA,
    ],
)(_prop_body)


def kernel(user_table, item_table, feat0, feat1, feat2,
           enc0_W, enc0_b, enc0_g, enc0_beta,
           enc1_W, enc1_b, enc1_g, enc1_beta,
           enc2_W, enc2_b, enc2_g, enc2_beta,
           att_W1, att_b1, att_W2, att_b2,
           adj_row, adj_col, adj_val):
    r1 = lambda v: v.reshape(1, -1)
    m2sum, usum = _user_stats(feat2, user_table, enc2_W, r1(enc2_b),
                              r1(enc2_g), r1(enc2_beta))
    aW2 = jnp.pad(att_W2, ((0, 0), (0, 128 - 3)))
    ab2 = jnp.pad(att_b2, (0, 128 - 3), constant_values=-1e30).reshape(1, -1)
    item_emb = _item_fuse(
        feat0, feat1, item_table,
        enc0_W, r1(enc0_b), r1(enc0_g), r1(enc0_beta),
        enc1_W, r1(enc1_b), r1(enc1_g), r1(enc1_beta),
        att_W1[:D], att_W1[D:], r1(att_b1), aW2, ab2, m2sum, usum)

    z = jnp.zeros((PAD, D), _f32)
    e0 = jnp.concatenate([user_table, z, item_emb, z], axis=0)

    npad = E_IDX - E
    rows2d = jnp.pad(adj_row.astype(_i32), (0, npad)).reshape(-1, EPC)
    cols2d = jnp.pad(adj_col.astype(_i32), (0, npad)).reshape(-1, EPC)
    vals1d = jnp.pad(adj_val, (0, npad))

    e1 = _propagate(e0, rows2d, cols2d, vals1d)
    e2 = _propagate(e1, rows2d, cols2d, vals1d)

    u_out = _mean3(user_table, e1[:NU], e2[:NU])
    i_out = _mean3(item_emb, e1[HALF_PAD:HALF_PAD + NI],
                   e2[HALF_PAD:HALF_PAD + NI])
    return u_out, i_out


# speedup vs baseline: 3.2123x; 1.4149x over previous
import functools

import jax
import jax.numpy as jnp
from jax import lax
from jax.experimental import pallas as pl
from jax.experimental.pallas import tpu as pltpu
from jax.experimental.pallas import tpu_sc as plsc

NU = 25000
NI = 25000
NN = NU + NI
D = 64
E = 800000

NC = 2
NS = 16
HALF = 25000
PAD = 88
HALF_PAD = HALF + PAD
NPAD = 2 * HALF_PAD
RPT = HALF_PAD // NS

EPC = 128
GPS = 8
SUPER = EPC * GPS
E_PAD = 819200
EPT = E_PAD // NS
NSUP = EPT // SUPER
E_IDX = E_PAD + 2 * SUPER

_f32 = jnp.float32
_i32 = jnp.int32


UB = 1000
IB = 1000


def _ln_leaky(h, g, beta):
    mu = jnp.mean(h, axis=-1, keepdims=True)
    var = jnp.mean((h - mu) ** 2, axis=-1, keepdims=True)
    h = (h - mu) * lax.rsqrt(var + 1e-5) * g + beta
    return jnp.where(h >= 0, h, 0.2 * h)


def _user_stats_body(feat2, utab, W, b, g, beta, m2sum, usum):
    i = pl.program_id(0)
    h = jnp.dot(feat2[...], W[...], preferred_element_type=_f32) + b[...]
    h = _ln_leaky(h, g[...], beta[...])

    @pl.when(i == 0)
    def _():
        m2sum[...] = jnp.zeros_like(m2sum)
        usum[...] = jnp.zeros_like(usum)

    m2sum[...] += jnp.sum(h, axis=0, keepdims=True)
    usum[...] += jnp.sum(utab[...], axis=0, keepdims=True)


def _user_stats(feat2, utab, W, b, g, beta):
    grid = (NU // UB,)
    full = lambda *s: pl.BlockSpec(s, lambda i: (0,) * len(s))
    return pl.pallas_call(
        _user_stats_body,
        grid=grid,
        in_specs=[
            pl.BlockSpec((UB, 128), lambda i: (i, 0)),
            pl.BlockSpec((UB, D), lambda i: (i, 0)),
            full(128, D), full(1, D), full(1, D), full(1, D),
        ],
        out_specs=(full(1, D), full(1, D)),
        out_shape=(jax.ShapeDtypeStruct((1, D), _f32),
                   jax.ShapeDtypeStruct((1, D), _f32)),
    )(feat2, utab, W, b, g, beta)


def _item_fuse_body(feat0, feat1, itab,
                    W0, b0, g0, be0, W1, b1, g1, be1,
                    aW1u, aW1i, ab1, aW2, ab2, m2sum, usum, out):
    m0 = _ln_leaky(jnp.dot(feat0[...], W0[...], preferred_element_type=_f32)
                   + b0[...], g0[...], be0[...])
    m1 = _ln_leaky(jnp.dot(feat1[...], W1[...], preferred_element_type=_f32)
                   + b1[...], g1[...], be1[...])
    m2m = m2sum[...] * (1.0 / NU)
    um = usum[...] * (1.0 / NU)
    it = itab[...]
    h = jnp.tanh(jnp.dot(it, aW1i[...], preferred_element_type=_f32)
                 + jnp.dot(um, aW1u[...], preferred_element_type=_f32)
                 + ab1[...])
    logits = jnp.dot(h, aW2[...], preferred_element_type=_f32) + ab2[...]
    mx = jnp.max(logits, axis=-1, keepdims=True)
    ex = jnp.exp(logits - mx)
    sm = ex / jnp.sum(ex, axis=-1, keepdims=True)
    out[...] = (it + sm[:, 0:1] * m0 + sm[:, 1:2] * m1 + sm[:, 2:3] * m2m)


def _item_fuse(feat0, feat1, itab, W0, b0, g0, be0, W1, b1, g1, be1,
               aW1u, aW1i, ab1, aW2, ab2, m2sum, usum):
    grid = (NI // IB,)
    full = lambda *s: pl.BlockSpec(s, lambda i: (0,) * len(s))
    return pl.pallas_call(
        _item_fuse_body,
        grid=grid,
        in_specs=[
            pl.BlockSpec((IB, 512), lambda i: (i, 0)),
            pl.BlockSpec((IB, 384), lambda i: (i, 0)),
            pl.BlockSpec((IB, D), lambda i: (i, 0)),
            full(512, D), full(1, D), full(1, D), full(1, D),
            full(384, D), full(1, D), full(1, D), full(1, D),
            full(D, D), full(D, D), full(1, D),
            full(D, 128), full(1, 128),
            full(1, D), full(1, D),
        ],
        out_specs=pl.BlockSpec((IB, D), lambda i: (i, 0)),
        out_shape=jax.ShapeDtypeStruct((NI, D), _f32),
    )(feat0, feat1, itab, W0, b0, g0, be0, W1, b1, g1, be1,
      aW1u, aW1i, ab1, aW2, ab2, m2sum, usum)


def _mean3_body(a, b, c, out):
    out[...] = (a[...] + b[...] + c[...]) * (1.0 / 3.0)


def _mean3(a, b, c):
    grid = (a.shape[0] // UB,)
    return pl.pallas_call(
        _mean3_body,
        grid=grid,
        in_specs=[pl.BlockSpec((UB, D), lambda i: (i, 0))] * 3,
        out_specs=pl.BlockSpec((UB, D), lambda i: (i, 0)),
        out_shape=jax.ShapeDtypeStruct(a.shape, _f32),
    )(a, b, c)


def _prop_body(emb, rows2d, cols2d, vals1d, out,
               acc, ridx0, cidx0, valv0, ridx1, cidx1, valv1,
               rbuf0, rbuf1, sidx,
               semI0, semI1, semG0, semG1, semS0, semS1):
    c = lax.axis_index("c")
    s = lax.axis_index("s")
    base = c * HALF
    rbufs = (rbuf0, rbuf1)
    semG = (semG0, semG1)
    semS = (semS0, semS1)
    slots = ((ridx0, cidx0, valv0, semI0), (ridx1, cidx1, valv1, semI1))
    zv = jnp.zeros((16,), _f32)
    civ = lax.iota(_i32, 16)
    ebase = s * EPT

    def zb(i, carry):
        for q in range(D // 16):
            rbuf0[i, pl.ds(16 * q, 16)] = zv
            rbuf1[i, pl.ds(16 * q, 16)] = zv
        return carry

    lax.fori_loop(0, EPC, zb, 0)
    for b in range(2):
        def sin(q, carry, _b=b):
            sidx[_b, pl.ds(16 * q, 16)] = jnp.full((16,), HALF, _i32) + civ
            return carry
        lax.fori_loop(0, EPC // 16, sin, 0)

    nfull = RPT // EPC
    rem = RPT - nfull * EPC
    for z in range(nfull):
        pltpu.sync_copy(rbuf0, acc.at[pl.ds(s * RPT + z * EPC, EPC)])
    if rem:
        pltpu.sync_copy(rbuf0.at[pl.ds(0, rem)],
                        acc.at[pl.ds(s * RPT + nfull * EPC, rem)])
    plsc.subcore_barrier()

    def issue_idx(m, slot):
        ridx_r, cidx_r, valv_r, semI = slot
        g0 = pl.multiple_of((ebase + m * SUPER) // EPC, GPS)
        e0_ = pl.multiple_of(ebase + m * SUPER, SUPER)
        pltpu.async_copy(rows2d.at[pl.ds(g0, GPS)], ridx_r, semI)
        pltpu.async_copy(cols2d.at[pl.ds(g0, GPS)], cidx_r, semI)
        pltpu.async_copy(vals1d.at[pl.ds(e0_, SUPER)], valv_r, semI)

    def wait_idx(m, slot):
        ridx_r, cidx_r, valv_r, semI = slot
        g0 = pl.multiple_of((ebase + m * SUPER) // EPC, GPS)
        e0_ = pl.multiple_of(ebase + m * SUPER, SUPER)
        pltpu.make_async_copy(rows2d.at[pl.ds(g0, GPS)], ridx_r, semI).wait()
        pltpu.make_async_copy(cols2d.at[pl.ds(g0, GPS)], cidx_r, semI).wait()
        pltpu.make_async_copy(vals1d.at[pl.ds(e0_, SUPER)], valv_r, semI).wait()

    def remap_copy(slot, g, b):
        ridx_r, cidx_r, valv_r, _ = slot

        @functools.partial(plsc.parallel_loop, 0, EPC // 16)
        def rq(q):
            sl = pl.ds(16 * q, 16)
            cid = cidx_r[g, sl]
            cidx_r[g, sl] = jnp.where(cid >= HALF, cid + PAD, cid)
            rid = ridx_r[g, sl]
            inh = (rid >= base) & (rid < base + HALF)
            dummy = jnp.full((16,), HALF, _i32) + 16 * (q & 3) + civ
            lloc = jnp.where(inh, rid - base, dummy)
            ridx_r[g, sl] = lloc
            sidx[b, sl] = lloc
            fl = pl.ds(g * EPC + 16 * q, 16)
            vv = valv_r[fl]
            valv_r[fl] = jnp.where(inh, vv, zv)

    def mul(slot, g, b):
        valv_r = slot[2]
        rb = rbufs[b]

        @functools.partial(plsc.parallel_loop, 0, EPC // 16)
        def mbody(j16):
            jb = j16 * 16
            v16 = valv_r[pl.ds(g * EPC + jb, 16)]
            for i in range(16):
                vi = v16[i]
                for q in range(D // 16):
                    sl2 = pl.ds(16 * q, 16)
                    rb[jb + i, sl2] = rb[jb + i, sl2] * vi

    def do_super(m, slot, cs):
        wait_idx(m, slot)
        cg = [None, None]
        for g in (0, 1):
            b = g & 1
            if cs[b] is not None:
                cs[b].wait()
            remap_copy(slot, g, b)
            cg[b] = pltpu.async_copy(emb.at[slot[1].at[g]], rbufs[b], semG[b])
        for g in range(GPS):
            b = g & 1
            cg[b].wait()
            mul(slot, g, b)
            cs[b] = pltpu.async_copy(rbufs[b], acc.at[sidx.at[b]], semS[b],
                                     add=True)
            if g + 2 < GPS:
                cs[b].wait()
                cs[b] = None
                remap_copy(slot, g + 2, b)
                cg[b] = pltpu.async_copy(emb.at[slot[1].at[g + 2]],
                                         rbufs[b], semG[b])
        issue_idx(m + 2, slot)
        return cs

    issue_idx(0, slots[0])
    issue_idx(1, slots[1])

    def body(i, carry):
        cs = do_super(2 * i, slots[0], [None, None])
        cs = do_super(2 * i + 1, slots[1], cs)
        cs[0].wait()
        cs[1].wait()
        return carry

    lax.fori_loop(0, NSUP // 2, body, 0)

    wait_idx(NSUP, slots[0])
    wait_idx(NSUP + 1, slots[1])
    plsc.subcore_barrier()
    pltpu.sync_copy(acc.at[pl.ds(s * RPT, RPT)],
                    out.at[pl.ds(c * HALF_PAD + s * RPT, RPT)])


_propagate = functools.partial(
    pl.kernel,
    out_type=jax.ShapeDtypeStruct((NPAD, D), _f32),
    mesh=plsc.VectorSubcoreMesh(core_axis_name="c", subcore_axis_name="s"),
    compiler_params=pltpu.CompilerParams(use_tc_tiling_on_sc=False),
    scratch_types=[
        pltpu.VMEM_SHARED((HALF_PAD, D), _f32),
        pltpu.VMEM((GPS, EPC), _i32),
        pltpu.VMEM((GPS, EPC), _i32),
        pltpu.VMEM((SUPER,), _f32),
        pltpu.VMEM((GPS, EPC), _i32),
        pltpu.VMEM((GPS, EPC), _i32),
        pltpu.VMEM((SUPER,), _f32),
        pltpu.VMEM((EPC, D), _f32),
        pltpu.VMEM((EPC, D), _f32),
        pltpu.VMEM((2, EPC), _i32),
        pltpu.SemaphoreType.DMA,
        pltpu.SemaphoreType.DMA,
        pltpu.SemaphoreType.DMA,
        pltpu.SemaphoreType.DMA,
        pltpu.SemaphoreType.DMA,
        pltpu.SemaphoreType.DMA,
    ],
)(_prop_body)


def kernel(user_table, item_table, feat0, feat1, feat2,
           enc0_W, enc0_b, enc0_g, enc0_beta,
           enc1_W, enc1_b, enc1_g, enc1_beta,
           enc2_W, enc2_b, enc2_g, enc2_beta,
           att_W1, att_b1, att_W2, att_b2,
           adj_row, adj_col, adj_val):
    r1 = lambda v: v.reshape(1, -1)
    m2sum, usum = _user_stats(feat2, user_table, enc2_W, r1(enc2_b),
                              r1(enc2_g), r1(enc2_beta))
    aW2 = jnp.pad(att_W2, ((0, 0), (0, 128 - 3)))
    ab2 = jnp.pad(att_b2, (0, 128 - 3), constant_values=-1e30).reshape(1, -1)
    item_emb = _item_fuse(
        feat0, feat1, item_table,
        enc0_W, r1(enc0_b), r1(enc0_g), r1(enc0_beta),
        enc1_W, r1(enc1_b), r1(enc1_g), r1(enc1_beta),
        att_W1[:D], att_W1[D:], r1(att_b1), aW2, ab2, m2sum, usum)

    z = jnp.zeros((PAD, D), _f32)
    e0 = jnp.concatenate([user_table, z, item_emb, z], axis=0)

    npad = E_IDX - E
    rows2d = jnp.pad(adj_row.astype(_i32), (0, npad)).reshape(-1, EPC)
    cols2d = jnp.pad(adj_col.astype(_i32), (0, npad)).reshape(-1, EPC)
    vals1d = jnp.pad(adj_val, (0, npad))

    e1 = _propagate(e0, rows2d, cols2d, vals1d)
    e2 = _propagate(e1, rows2d, cols2d, vals1d)

    u_out = _mean3(user_table, e1[:NU], e2[:NU])
    i_out = _mean3(item_emb, e1[HALF_PAD:HALF_PAD + NI],
                   e2[HALF_PAD:HALF_PAD + NI])
    return u_out, i_out
